# baseline (speedup 1.0000x reference)
import functools

import jax
import jax.numpy as jnp
from jax import lax
from jax.experimental import pallas as pl
from jax.experimental.pallas import tpu as pltpu
from jax.experimental.pallas import tpu_sc as plsc


def _sc_gather(emb, ids, n_chunks, chunk):
    NW = ids.shape[0]
    n_per_w = n_chunks * chunk
    N = NW * n_per_w
    E = emb.shape[1]
    mesh = plsc.VectorSubcoreMesh(core_axis_name="c", subcore_axis_name="s")
    num_cores = 2

    @functools.partial(
        pl.kernel,
        mesh=mesh,
        out_type=jax.ShapeDtypeStruct((N, E), jnp.float32),
        compiler_params=pltpu.CompilerParams(use_tc_tiling_on_sc=False),
        scratch_types=[
            pltpu.VMEM((n_chunks, chunk), jnp.int32),
            pltpu.VMEM((n_per_w, E), jnp.float32),
            pltpu.SemaphoreType.DMA,
        ],
    )
    def gather_kernel(emb_hbm, idx_hbm, out_hbm, idx_v, rows_v, sem):
        wid = lax.axis_index("s") * num_cores + lax.axis_index("c")
        base = wid * n_per_w
        pltpu.sync_copy(idx_hbm.at[wid], idx_v)

        def issue(j, carry):
            pltpu.async_copy(
                emb_hbm.at[idx_v.at[j]],
                rows_v.at[pl.ds(j * chunk, chunk)],
                sem,
            )
            return carry

        lax.fori_loop(0, n_chunks, issue, 0)
        pltpu.make_async_copy(
            out_hbm.at[pl.ds(base, n_per_w)], rows_v, sem
        ).wait()
        pltpu.sync_copy(rows_v, out_hbm.at[pl.ds(base, n_per_w)])

    return gather_kernel(emb, ids)


def _lstm_body(S, B, E, H, xseq_ref, wih_ref, whh_ref, b_ref, wfc_ref,
               bfc_ref, y_ref, h_scr, c_scr):
    h_scr[...] = jnp.zeros((B, H), jnp.float32)
    c_scr[...] = jnp.zeros((B, H), jnp.float32)

    def step(t, carry):
        x_t = xseq_ref[t]
        h = h_scr[...]
        c = c_scr[...]
        gates = (
            jnp.dot(x_t, wih_ref[...], preferred_element_type=jnp.float32)
            + jnp.dot(h, whh_ref[...], preferred_element_type=jnp.float32)
            + b_ref[...]
        )
        i = jax.nn.sigmoid(gates[:, 0:H])
        f = jax.nn.sigmoid(gates[:, H:2 * H])
        g = jnp.tanh(gates[:, 2 * H:3 * H])
        o = jax.nn.sigmoid(gates[:, 3 * H:4 * H])
        c_new = f * c + i * g
        h_new = o * jnp.tanh(c_new)
        h_scr[...] = h_new
        c_scr[...] = c_new
        return carry

    lax.fori_loop(0, S, step, 0)
    y_ref[...] = (
        jnp.dot(h_scr[...], wfc_ref[...], preferred_element_type=jnp.float32)
        + bfc_ref[...]
    )


def _tc_lstm(xseq, wihT, whhT, bias, wfcT, bfc):
    S, B, E = xseq.shape
    H = whhT.shape[0]
    CLS = wfcT.shape[1]
    body = functools.partial(_lstm_body, S, B, E, H)
    return pl.pallas_call(
        body,
        out_shape=jax.ShapeDtypeStruct((B, CLS), jnp.float32),
        scratch_shapes=[
            pltpu.VMEM((B, H), jnp.float32),
            pltpu.VMEM((B, H), jnp.float32),
        ],
    )(xseq, wihT, whhT, bias, wfcT, bfc)


def kernel(x, emb, W_ih, W_hh, b_ih, b_hh, W_fc, b_fc):
    B, S = x.shape
    E = emb.shape[1]
    H = W_hh.shape[1]
    NW = 32
    chunk = 100
    N = B * S
    n_per_w = N // NW
    n_chunks = n_per_w // chunk
    assert n_chunks * chunk * NW == N

    ids = jnp.transpose(x).reshape(NW, n_chunks, chunk).astype(jnp.int32)
    gathered = _sc_gather(emb, ids, n_chunks, chunk)
    xseq = gathered.reshape(S, B, E)

    wihT = jnp.transpose(W_ih)
    whhT = jnp.transpose(W_hh)
    bias = (b_ih + b_hh).reshape(1, 4 * H)
    wfcT = jnp.transpose(W_fc)
    bfc = b_fc.reshape(1, -1)

    return _tc_lstm(xseq, wihT, whhT, bias, wfcT, bfc)
